# baseline (speedup 1.0000x reference)
import functools
import math

import jax
import jax.numpy as jnp
from jax import lax
from jax.experimental import pallas as pl
from jax.experimental.pallas import tpu as pltpu
from jax.experimental.pallas import tpu_sc as plsc

L = 16
STAGE = 1024
SUB = 256


def _make_kernel(VP, D, B, NW, NC):
    per_w = B // NW
    nstage = per_w // STAGE
    nsub = STAGE // SUB
    ngrp = SUB // L
    vregs_per_row = D // L

    mesh = plsc.VectorSubcoreMesh(core_axis_name="c", subcore_axis_name="s")

    @functools.partial(
        pl.kernel,
        out_type=jax.ShapeDtypeStruct((B, D), jnp.float32),
        mesh=mesh,
        scratch_types=[
            pltpu.VMEM((STAGE,), jnp.int32),
            pltpu.VMEM((STAGE,), jnp.int32),
            pltpu.VMEM((SUB, 2 * D), jnp.float32),
            pltpu.VMEM((SUB, D), jnp.float32),
            pltpu.SemaphoreType.DMA,
        ],
    )
    def emb(tbl_hbm, idx_hbm, par_hbm, out_hbm, idx_v, par_v, rows_v, outb_v,
            gsem):
        wid = lax.axis_index("s") * NC + lax.axis_index("c")
        wbase = pl.multiple_of(wid * per_w, STAGE)

        def stage_body(s, carry):
            t0 = pl.multiple_of(wbase + s * STAGE, STAGE)
            pltpu.sync_copy(idx_hbm.at[pl.ds(t0, STAGE)], idx_v)
            pltpu.sync_copy(par_hbm.at[pl.ds(t0, STAGE)], par_v)

            def sub_body(u, carry2):
                o0 = pl.multiple_of(u * SUB, SUB)
                cps = [
                    pltpu.async_copy(
                        tbl_hbm.at[idx_v[pl.ds(o0 + k * L, L)]],
                        rows_v.at[pl.ds(k * L, L)],
                        gsem,
                    )
                    for k in range(ngrp)
                ]
                for cp in cps:
                    cp.wait()

                def grp_body(g, carry3):
                    gof = pl.multiple_of(g * L, L)
                    pv = par_v[pl.ds(o0 + gof, L)]
                    for t in range(L):
                        p = pv[t]
                        row = gof + t
                        for l in range(vregs_per_row):
                            outb_v[row, pl.ds(l * L, L)] = (
                                rows_v[row, pl.ds(p + l * L, L)] * 8.0
                            )
                    return carry3

                lax.fori_loop(0, ngrp, grp_body, 0)

                pltpu.sync_copy(
                    outb_v, out_hbm.at[pl.ds(t0 + u * SUB, SUB)]
                )
                return carry2

            lax.fori_loop(0, nsub, sub_body, carry)
            return carry

        lax.fori_loop(0, nstage, stage_body, 0)

    return emb


def kernel(x, lut):
    B0, B1 = x.shape
    V, D = lut.shape
    B = B0 * B1
    info = plsc.get_sparse_core_info()
    NC, NS = info.num_cores, info.num_subcores
    NW = NC * NS
    if x.dtype != jnp.int32:
        x = x.astype(jnp.int32)
    tbl = lut.reshape(V // 2, 2 * D)
    pair = (x >> 1).reshape(B)
    par = ((x & 1) << 6).reshape(B)
    out = _make_kernel(V // 2, D, B, NW, NC)(tbl, pair, par)
    return out.reshape(B0, B1, D)
